# baseline (speedup 1.0000x reference)
import jax
import jax.numpy as jnp
from jax import lax
from jax.experimental import pallas as pl
from jax.experimental.pallas import tpu as pltpu
from jax.experimental.pallas import tpu_sc as plsc

B, T, D = 4, 2048, 1024
B_TC = 3
B_SC = B - B_TC
NC, NS, L = 2, 16, 16
NW = NC * NS
TPW = T // NW
CT = 16
NCH = TPW // CT
NK = NCH * B_SC
NSLOT = 5
AHEAD = 3


def _sc_body(x_hbm, pos_hbm, out_hbm, xbuf, pos_buf, sin, sin2, sout, sout2,
             spos):
    wid = lax.axis_index("s") * NC + lax.axis_index("c")
    t_base = wid * TPW

    def fire_pos(c, slot):
        pltpu.async_copy(
            pos_hbm.at[pl.ds(t_base + c * CT, CT)], pos_buf.at[slot],
            spos.at[slot])

    H = CT // 2

    def fire_in(k, slot):
        c, b = k // B_SC, B_TC + k % B_SC
        t0 = t_base + c * CT
        pltpu.async_copy(
            x_hbm.at[b, pl.ds(t0, H)], xbuf.at[slot, pl.ds(0, H)],
            sin.at[slot])
        pltpu.async_copy(
            x_hbm.at[b, pl.ds(t0 + H, H)], xbuf.at[slot, pl.ds(H, H)],
            sin2.at[slot])

    def wait_in(slot):
        pltpu.make_async_copy(
            x_hbm.at[0, pl.ds(0, H)], xbuf.at[slot, pl.ds(0, H)],
            sin.at[slot]).wait()
        pltpu.make_async_copy(
            x_hbm.at[0, pl.ds(0, H)], xbuf.at[slot, pl.ds(H, H)],
            sin2.at[slot]).wait()

    def fire_out(k, slot):
        c, b = k // B_SC, k % B_SC
        t0 = t_base + c * CT
        pltpu.async_copy(
            xbuf.at[slot, pl.ds(0, H)], out_hbm.at[b, pl.ds(t0, H)],
            sout.at[slot])
        pltpu.async_copy(
            xbuf.at[slot, pl.ds(H, H)], out_hbm.at[b, pl.ds(t0 + H, H)],
            sout2.at[slot])

    def wait_out(slot):
        pltpu.make_async_copy(
            xbuf.at[slot, pl.ds(0, H)], out_hbm.at[0, pl.ds(0, H)],
            sout.at[slot]).wait()
        pltpu.make_async_copy(
            xbuf.at[slot, pl.ds(H, H)], out_hbm.at[0, pl.ds(0, H)],
            sout2.at[slot]).wait()

    def wait_pos(slot):
        pltpu.make_async_copy(
            pos_hbm.at[pl.ds(0, CT)], pos_buf.at[slot], spos.at[slot]).wait()

    fire_pos(0, 0)
    for k in range(AHEAD):
        fire_in(k, k % NSLOT)

    def body(k, _):
        c = k // B_SC
        s = k % NSLOT
        pc = c % 2

        @pl.when(k % B_SC == 0)
        def _():
            wait_pos(pc)

            @pl.when(c + 1 < NCH)
            def _():
                fire_pos(c + 1, (c + 1) % 2)

        wait_in(s)

        @plsc.parallel_loop(0, CT, 1, unroll=2)
        def _rows(i):
            for j in range(D // L):
                v = pos_buf[pc, i, pl.ds(j * L, L)]
                plsc.addupdate(xbuf.at[s, i, pl.ds(j * L, L)], v)

        fire_out(k, s)

        k2 = k + AHEAD

        @pl.when(k2 < NK)
        def _():
            s2 = k2 % NSLOT

            @pl.when(k2 >= NSLOT)
            def _():
                wait_out(s2)

            fire_in(k2, s2)

        return 0

    lax.fori_loop(0, NK, body, 0)
    for s in range(min(NSLOT, NK)):
        wait_out(s)


def _sc_kernel(x, pos_table):
    mesh = plsc.VectorSubcoreMesh(core_axis_name="c", subcore_axis_name="s")
    f = pl.kernel(
        _sc_body,
        out_type=jax.ShapeDtypeStruct((B_SC, T, D), jnp.float32),
        mesh=mesh,
        scratch_types=[
            pltpu.VMEM((NSLOT, CT, D), jnp.float32),
            pltpu.VMEM((2, CT, D), jnp.float32),
            pltpu.SemaphoreType.DMA((NSLOT,)),
            pltpu.SemaphoreType.DMA((NSLOT,)),
            pltpu.SemaphoreType.DMA((NSLOT,)),
            pltpu.SemaphoreType.DMA((NSLOT,)),
            pltpu.SemaphoreType.DMA((2,)),
        ],
    )
    return f(x, pos_table)


def _tc_body(x_ref, pos_ref, o_ref):
    o_ref[...] = x_ref[...] + pos_ref[...]


def _tc_kernel(x, pos_table):
    BT = 2048
    grid = (T // BT, B_TC)
    return pl.pallas_call(
        _tc_body,
        grid=grid,
        in_specs=[
            pl.BlockSpec((1, BT, D), lambda t, b: (b, t, 0)),
            pl.BlockSpec((BT, D), lambda t, b: (t, 0)),
        ],
        out_specs=pl.BlockSpec((1, BT, D), lambda t, b: (b, t, 0)),
        out_shape=jax.ShapeDtypeStruct((B_TC, T, D), x.dtype),
    )(x, pos_table)


def kernel(x, pos_table):
    out_sc = _sc_kernel(x, pos_table)
    out_tc = _tc_kernel(x, pos_table)
    return jnp.concatenate([out_tc, out_sc], axis=0)


# speedup vs baseline: 1.3522x; 1.3522x over previous
import jax
import jax.numpy as jnp
from jax import lax
from jax.experimental import pallas as pl
from jax.experimental.pallas import tpu as pltpu
from jax.experimental.pallas import tpu_sc as plsc

B, T, D = 4, 2048, 1024
NC, NS, L = 2, 16, 16
NW = NC * NS
TPW = T // NW
CT = 16
NCH = TPW // CT
NK = NCH * B
NSLOT = 5
AHEAD = 3


def _sc_body(x_hbm, pos_hbm, out_hbm, xbuf, pos_buf, sin, sin2, sout, sout2,
             spos):
    wid = lax.axis_index("s") * NC + lax.axis_index("c")
    t_base = wid * TPW

    def fire_pos(c, slot):
        pltpu.async_copy(
            pos_hbm.at[pl.ds(t_base + c * CT, CT)], pos_buf.at[slot],
            spos.at[slot])

    H = CT // 2

    def fire_in(k, slot):
        c, b = k // B, k % B
        t0 = t_base + c * CT
        pltpu.async_copy(
            x_hbm.at[b, pl.ds(t0, H)], xbuf.at[slot, pl.ds(0, H)],
            sin.at[slot])
        pltpu.async_copy(
            x_hbm.at[b, pl.ds(t0 + H, H)], xbuf.at[slot, pl.ds(H, H)],
            sin2.at[slot])

    def wait_in(slot):
        pltpu.make_async_copy(
            x_hbm.at[0, pl.ds(0, H)], xbuf.at[slot, pl.ds(0, H)],
            sin.at[slot]).wait()
        pltpu.make_async_copy(
            x_hbm.at[0, pl.ds(0, H)], xbuf.at[slot, pl.ds(H, H)],
            sin2.at[slot]).wait()

    def fire_out(k, slot):
        c, b = k // B, k % B
        t0 = t_base + c * CT
        pltpu.async_copy(
            xbuf.at[slot, pl.ds(0, H)], out_hbm.at[b, pl.ds(t0, H)],
            sout.at[slot])
        pltpu.async_copy(
            xbuf.at[slot, pl.ds(H, H)], out_hbm.at[b, pl.ds(t0 + H, H)],
            sout2.at[slot])

    def wait_out(slot):
        pltpu.make_async_copy(
            xbuf.at[slot, pl.ds(0, H)], out_hbm.at[0, pl.ds(0, H)],
            sout.at[slot]).wait()
        pltpu.make_async_copy(
            xbuf.at[slot, pl.ds(H, H)], out_hbm.at[0, pl.ds(0, H)],
            sout2.at[slot]).wait()

    def wait_pos(slot):
        pltpu.make_async_copy(
            pos_hbm.at[pl.ds(0, CT)], pos_buf.at[slot], spos.at[slot]).wait()

    fire_pos(0, 0)
    for k in range(AHEAD):
        fire_in(k, k % NSLOT)

    def body(k, _):
        c, b = k // B, k % B
        s = k % NSLOT
        pc = c % 2

        @pl.when(b == 0)
        def _():
            wait_pos(pc)

            @pl.when(c + 1 < NCH)
            def _():
                fire_pos(c + 1, (c + 1) % 2)

        wait_in(s)

        @plsc.parallel_loop(0, CT, 1, unroll=4)
        def _rows(i):
            for j in range(D // L):
                v = pos_buf[pc, i, pl.ds(j * L, L)]
                plsc.addupdate(xbuf.at[s, i, pl.ds(j * L, L)], v)

        fire_out(k, s)

        k2 = k + AHEAD

        @pl.when(k2 < NK)
        def _():
            s2 = k2 % NSLOT

            @pl.when(k2 >= NSLOT)
            def _():
                wait_out(s2)

            fire_in(k2, s2)

        return 0

    lax.fori_loop(0, NK, body, 0)
    for s in range(NSLOT):
        wait_out(s)


def _sc_kernel(x, pos_table):
    mesh = plsc.VectorSubcoreMesh(core_axis_name="c", subcore_axis_name="s")
    f = pl.kernel(
        _sc_body,
        out_type=jax.ShapeDtypeStruct((B, T, D), jnp.float32),
        mesh=mesh,
        scratch_types=[
            pltpu.VMEM((NSLOT, CT, D), jnp.float32),
            pltpu.VMEM((2, CT, D), jnp.float32),
            pltpu.SemaphoreType.DMA((NSLOT,)),
            pltpu.SemaphoreType.DMA((NSLOT,)),
            pltpu.SemaphoreType.DMA((NSLOT,)),
            pltpu.SemaphoreType.DMA((NSLOT,)),
            pltpu.SemaphoreType.DMA((2,)),
        ],
    )
    return f(x, pos_table)


def kernel(x, pos_table):
    return _sc_kernel(x, pos_table)


# speedup vs baseline: 1.3570x; 1.0035x over previous
import jax
import jax.numpy as jnp
from jax import lax
from jax.experimental import pallas as pl
from jax.experimental.pallas import tpu as pltpu
from jax.experimental.pallas import tpu_sc as plsc

B, T, D = 4, 2048, 1024
NC, NS, L = 2, 16, 16
NW = NC * NS
TPW = T // NW
CT = 16
NCH = TPW // CT
NK = NCH * B
NSLOT = 5
AHEAD = 3


def _sc_body(x_hbm, pos_hbm, out_hbm, xbuf, pos_buf, sin, sin2, sout, sout2,
             spos):
    wid = lax.axis_index("s") * NC + lax.axis_index("c")
    t_base = wid * TPW

    def fire_pos(c, slot):
        pltpu.async_copy(
            pos_hbm.at[pl.ds(t_base + c * CT, CT)], pos_buf.at[slot],
            spos.at[slot])

    H = CT // 2

    def fire_in(k, slot):
        c, b = k // B, k % B
        t0 = t_base + c * CT
        pltpu.async_copy(
            x_hbm.at[b, pl.ds(t0, H)], xbuf.at[slot, pl.ds(0, H)],
            sin.at[slot])
        pltpu.async_copy(
            x_hbm.at[b, pl.ds(t0 + H, H)], xbuf.at[slot, pl.ds(H, H)],
            sin2.at[slot])

    def wait_in(slot):
        pltpu.make_async_copy(
            x_hbm.at[0, pl.ds(0, H)], xbuf.at[slot, pl.ds(0, H)],
            sin.at[slot]).wait()
        pltpu.make_async_copy(
            x_hbm.at[0, pl.ds(0, H)], xbuf.at[slot, pl.ds(H, H)],
            sin2.at[slot]).wait()

    def fire_out(k, slot):
        c, b = k // B, k % B
        t0 = t_base + c * CT
        pltpu.async_copy(
            xbuf.at[slot, pl.ds(0, H)], out_hbm.at[b, pl.ds(t0, H)],
            sout.at[slot])
        pltpu.async_copy(
            xbuf.at[slot, pl.ds(H, H)], out_hbm.at[b, pl.ds(t0 + H, H)],
            sout2.at[slot])

    def wait_out(slot):
        pltpu.make_async_copy(
            xbuf.at[slot, pl.ds(0, H)], out_hbm.at[0, pl.ds(0, H)],
            sout.at[slot]).wait()
        pltpu.make_async_copy(
            xbuf.at[slot, pl.ds(H, H)], out_hbm.at[0, pl.ds(0, H)],
            sout2.at[slot]).wait()

    def wait_pos(slot):
        pltpu.make_async_copy(
            pos_hbm.at[pl.ds(0, CT)], pos_buf.at[slot], spos.at[slot]).wait()

    fire_pos(0, 0)
    for k in range(AHEAD):
        fire_in(k, k % NSLOT)

    def body(k, _):
        c, b = k // B, k % B
        s = k % NSLOT
        pc = c % 2

        @pl.when(b == 0)
        def _():
            wait_pos(pc)

            @pl.when(c + 1 < NCH)
            def _():
                fire_pos(c + 1, (c + 1) % 2)

        wait_in(s)

        @plsc.parallel_loop(0, CT, 1, unroll=2)
        def _rows(i):
            for j in range(D // L):
                v = pos_buf[pc, i, pl.ds(j * L, L)]
                plsc.addupdate(xbuf.at[s, i, pl.ds(j * L, L)], v)

        fire_out(k, s)

        k2 = k + AHEAD

        @pl.when(k2 < NK)
        def _():
            s2 = k2 % NSLOT

            @pl.when(k2 >= NSLOT)
            def _():
                wait_out(s2)

            fire_in(k2, s2)

        return 0

    lax.fori_loop(0, NK, body, 0)
    for s in range(NSLOT):
        wait_out(s)


def _sc_kernel(x, pos_table):
    mesh = plsc.VectorSubcoreMesh(core_axis_name="c", subcore_axis_name="s")
    f = pl.kernel(
        _sc_body,
        out_type=jax.ShapeDtypeStruct((B, T, D), jnp.float32),
        mesh=mesh,
        scratch_types=[
            pltpu.VMEM((NSLOT, CT, D), jnp.float32),
            pltpu.VMEM((2, CT, D), jnp.float32),
            pltpu.SemaphoreType.DMA((NSLOT,)),
            pltpu.SemaphoreType.DMA((NSLOT,)),
            pltpu.SemaphoreType.DMA((NSLOT,)),
            pltpu.SemaphoreType.DMA((NSLOT,)),
            pltpu.SemaphoreType.DMA((2,)),
        ],
    )
    return f(x, pos_table)


def kernel(x, pos_table):
    return _sc_kernel(x, pos_table)


# speedup vs baseline: 1.3599x; 1.0022x over previous
import jax
import jax.numpy as jnp
from jax import lax
from jax.experimental import pallas as pl
from jax.experimental.pallas import tpu as pltpu
from jax.experimental.pallas import tpu_sc as plsc

B, T, D = 4, 2048, 1024
NC, NS, L = 2, 16, 16
NW = NC * NS
TPW = T // NW
CT = 16
NCH = TPW // CT
NK = NCH * B
NSLOT = 5
AHEAD = 3


def _sc_body(x_hbm, pos_hbm, out_hbm, xbuf, pos_buf, sin, sin2, sout, sout2,
             spos):
    wid = lax.axis_index("s") * NC + lax.axis_index("c")
    t_base = wid * TPW

    def fire_pos(c, slot):
        pltpu.async_copy(
            pos_hbm.at[pl.ds(t_base + c * CT, CT)], pos_buf.at[slot],
            spos.at[slot])

    H = CT // 2

    def fire_in(k, slot):
        c, b = k // B, k % B
        t0 = t_base + c * CT
        pltpu.async_copy(
            x_hbm.at[b, pl.ds(t0, H)], xbuf.at[slot, pl.ds(0, H)],
            sin.at[slot])
        pltpu.async_copy(
            x_hbm.at[b, pl.ds(t0 + H, H)], xbuf.at[slot, pl.ds(H, H)],
            sin2.at[slot])

    def wait_in(slot):
        pltpu.make_async_copy(
            x_hbm.at[0, pl.ds(0, H)], xbuf.at[slot, pl.ds(0, H)],
            sin.at[slot]).wait()
        pltpu.make_async_copy(
            x_hbm.at[0, pl.ds(0, H)], xbuf.at[slot, pl.ds(H, H)],
            sin2.at[slot]).wait()

    def fire_out(k, slot):
        c, b = k // B, k % B
        t0 = t_base + c * CT
        pltpu.async_copy(
            xbuf.at[slot, pl.ds(0, H)], out_hbm.at[b, pl.ds(t0, H)],
            sout.at[slot])
        pltpu.async_copy(
            xbuf.at[slot, pl.ds(H, H)], out_hbm.at[b, pl.ds(t0 + H, H)],
            sout2.at[slot])

    def wait_out(slot):
        pltpu.make_async_copy(
            xbuf.at[slot, pl.ds(0, H)], out_hbm.at[0, pl.ds(0, H)],
            sout.at[slot]).wait()
        pltpu.make_async_copy(
            xbuf.at[slot, pl.ds(H, H)], out_hbm.at[0, pl.ds(0, H)],
            sout2.at[slot]).wait()

    def wait_pos(slot):
        pltpu.make_async_copy(
            pos_hbm.at[pl.ds(0, CT)], pos_buf.at[slot], spos.at[slot]).wait()

    fire_pos(0, 0)
    for k in range(AHEAD):
        fire_in(k, k % NSLOT)

    def body(k, _):
        c, b = k // B, k % B
        s = k % NSLOT
        pc = c % 2

        @pl.when(b == 0)
        def _():
            wait_pos(pc)

            @pl.when(c + 1 < NCH)
            def _():
                fire_pos(c + 1, (c + 1) % 2)

        wait_in(s)

        k2 = k + AHEAD

        @pl.when(k2 < NK)
        def _():
            s2 = k2 % NSLOT

            @pl.when(k2 >= NSLOT)
            def _():
                wait_out(s2)

            fire_in(k2, s2)

        @plsc.parallel_loop(0, CT, 1, unroll=2)
        def _rows(i):
            for j in range(D // L):
                v = pos_buf[pc, i, pl.ds(j * L, L)]
                plsc.addupdate(xbuf.at[s, i, pl.ds(j * L, L)], v)

        fire_out(k, s)

        return 0

    lax.fori_loop(0, NK, body, 0)
    for s in range(NSLOT):
        wait_out(s)


def _sc_kernel(x, pos_table):
    mesh = plsc.VectorSubcoreMesh(core_axis_name="c", subcore_axis_name="s")
    f = pl.kernel(
        _sc_body,
        out_type=jax.ShapeDtypeStruct((B, T, D), jnp.float32),
        mesh=mesh,
        scratch_types=[
            pltpu.VMEM((NSLOT, CT, D), jnp.float32),
            pltpu.VMEM((2, CT, D), jnp.float32),
            pltpu.SemaphoreType.DMA((NSLOT,)),
            pltpu.SemaphoreType.DMA((NSLOT,)),
            pltpu.SemaphoreType.DMA((NSLOT,)),
            pltpu.SemaphoreType.DMA((NSLOT,)),
            pltpu.SemaphoreType.DMA((2,)),
        ],
    )
    return f(x, pos_table)


def kernel(x, pos_table):
    return _sc_kernel(x, pos_table)
